# baseline (speedup 1.0000x reference)
import functools

import jax
import jax.numpy as jnp
from jax import lax
from jax.experimental import pallas as pl
from jax.experimental.pallas import tpu as pltpu
from jax.experimental.pallas import tpu_sc as plsc

N = 10000
E = 160000
D = 256
_RB = 2000

_NC, _NS = 2, 16
_NW = _NC * _NS
_RW = 157
_NSUB = 2 * _NW
_NPAD = _NSUB * _RW
_CE = 4000
_NCHUNK = E // _CE
_VPC = _CE // 16
_G = 64
_MCAP = _CE + 128
_FMAX = 3.4028235e38


def _pool_body(x_ref, w_ref, b_ref, o_ref):
    acc = jax.lax.dot_general(
        x_ref[...], w_ref[...], (((1,), (1,)), ((), ())),
        preferred_element_type=jnp.float32)
    o_ref[...] = jnp.maximum(acc + b_ref[...], 0.0)


def _pool_matmul(x, W, b):
    return pl.pallas_call(
        _pool_body,
        grid=(N // _RB,),
        in_specs=[
            pl.BlockSpec((_RB, D), lambda i: (i, 0)),
            pl.BlockSpec((D, D), lambda i: (0, 0)),
            pl.BlockSpec((1, D), lambda i: (0, 0)),
        ],
        out_specs=pl.BlockSpec((_RB, D), lambda i: (i, 0)),
        out_shape=jax.ShapeDtypeStruct((N, D), jnp.float32),
    )(x, W, b.reshape(1, D))


def _out_body(x_ref, ws_ref, e_ref, we_ref, b_ref, o_ref, *, relu):
    acc = jax.lax.dot_general(
        x_ref[...], ws_ref[...], (((1,), (1,)), ((), ())),
        preferred_element_type=jnp.float32)
    acc = acc + jax.lax.dot_general(
        e_ref[...], we_ref[...], (((1,), (1,)), ((), ())),
        preferred_element_type=jnp.float32)
    acc = acc + b_ref[...]
    if relu:
        acc = jnp.maximum(acc, 0.0)
    o_ref[...] = acc


def _out_matmul(x, Wself, eve, Weve, b, relu):
    return pl.pallas_call(
        functools.partial(_out_body, relu=relu),
        grid=(N // _RB,),
        in_specs=[
            pl.BlockSpec((_RB, D), lambda i: (i, 0)),
            pl.BlockSpec((D, D), lambda i: (0, 0)),
            pl.BlockSpec((_RB, D), lambda i: (i, 0)),
            pl.BlockSpec((D, D), lambda i: (0, 0)),
            pl.BlockSpec((1, D), lambda i: (0, 0)),
        ],
        out_specs=pl.BlockSpec((_RB, D), lambda i: (i, 0)),
        out_shape=jax.ShapeDtypeStruct((N, D), jnp.float32),
    )(x, Wself, eve, Weve, b.reshape(1, D))


def _sc_eve_body(h_hbm, src_hbm, dst_hbm, w_hbm, out_hbm,
                 amax, amin, dstb, srcb, msrc, mdloc, rows, wr, sem):
    wid = lax.axis_index("s") * _NC + lax.axis_index("c")

    pltpu.sync_copy(w_hbm, wr)
    w0 = wr[0, :]
    w1 = wr[1, :]
    wb = wr[2, :]

    def _initm(i, _):
        msrc[pl.ds(i * 16, 16)] = jnp.zeros((16,), jnp.int32)
        mdloc[pl.ds(i * 16, 16)] = jnp.zeros((16,), jnp.int32)
        return 0
    lax.fori_loop(0, _MCAP // 16, _initm, 0)

    def _subrange(r, _):
        sid = wid * 2 + r
        lo = sid * _RW

        def _inita(i, _):
            amax[pl.ds(i * 16, 16)] = jnp.full((16,), -1.0, jnp.float32)
            amin[pl.ds(i * 16, 16)] = jnp.full((16,), _FMAX, jnp.float32)
            return 0
        lax.fori_loop(0, (_RW + 1) * D // 16, _inita, 0)

        def _chunk(c, _):
            pltpu.sync_copy(dst_hbm.at[pl.ds(c * _CE, _CE)], dstb)
            pltpu.sync_copy(src_hbm.at[pl.ds(c * _CE, _CE)], srcb)

            trash = jax.lax.iota(jnp.int32, 16) + (_MCAP - 16)
            lov = jnp.broadcast_to(lo, (16,)).astype(jnp.int32)
            hiv = lov + _RW

            def _scan(v, cnt):
                dvec = dstb[pl.ds(v * 16, 16)]
                svec = srcb[pl.ds(v * 16, 16)]
                m = (dvec >= lov) & (dvec < hiv)
                mi = m.astype(jnp.int32)
                cs = plsc.cumsum(mi)
                cntv = jnp.broadcast_to(cnt, (16,)).astype(jnp.int32)
                pos = jnp.where(m, cntv + cs - mi, trash)
                plsc.store_scatter(msrc, [pos], svec)
                plsc.store_scatter(mdloc, [pos], dvec - lov)
                return cnt + cs[15]

            cnt = lax.fori_loop(0, _VPC, _scan, jnp.int32(0))
            nb = (cnt + _G - 1) // _G

            def _batch(b, _):
                pltpu.async_copy(h_hbm.at[msrc.at[pl.ds(b * _G, _G)]],
                                 rows, sem).wait()

                def _edge(j, _):
                    @pl.when(b * _G + j < cnt)
                    def _():
                        dl = mdloc[pl.ds(b * _G + j, 16)][0]
                        base = dl * D
                        for k in range(D // 16):
                            rv = rows[j, pl.ds(k * 16, 16)]
                            off = base + k * 16
                            amax[pl.ds(off, 16)] = jnp.maximum(
                                amax[pl.ds(off, 16)], rv)
                            amin[pl.ds(off, 16)] = jnp.minimum(
                                amin[pl.ds(off, 16)], rv)
                    return 0
                lax.fori_loop(0, _G, _edge, 0)
                return 0

            lax.fori_loop(0, nb, _batch, 0)
            return 0

        lax.fori_loop(0, _NCHUNK, _chunk, 0)

        def _fin(i, _):
            off = i * 16
            mx = amax[pl.ds(off, 16)]
            mn = amin[pl.ds(off, 16)]
            ne = mx < 0.0
            mx = jnp.where(ne, 0.0, mx)
            mn = jnp.where(ne, 0.0, mn)
            amax[pl.ds(off, 16)] = jnp.maximum(w0 * mx + w1 * mn + wb, 0.0)
            return 0
        lax.fori_loop(0, _RW * D // 16, _fin, 0)
        pltpu.sync_copy(amax.at[pl.ds(0, _RW * D)],
                        out_hbm.at[pl.ds(lo * D, _RW * D)])
        return 0

    lax.fori_loop(0, 2, _subrange, 0)


def _sc_eve(h, src, dst, dww, dwb):
    w = jnp.stack([jnp.full((16,), dww[0], jnp.float32),
                   jnp.full((16,), dww[1], jnp.float32),
                   jnp.full((16,), dwb[0], jnp.float32)])
    mesh = plsc.VectorSubcoreMesh(core_axis_name="c", subcore_axis_name="s",
                                  num_cores=_NC, num_subcores=_NS)
    run = pl.kernel(
        _sc_eve_body,
        out_type=jax.ShapeDtypeStruct((_NPAD * D,), jnp.float32),
        mesh=mesh,
        scratch_types=[
            pltpu.VMEM(((_RW + 1) * D,), jnp.float32),
            pltpu.VMEM(((_RW + 1) * D,), jnp.float32),
            pltpu.VMEM((_CE,), jnp.int32),
            pltpu.VMEM((_CE,), jnp.int32),
            pltpu.VMEM((_MCAP,), jnp.int32),
            pltpu.VMEM((_MCAP,), jnp.int32),
            pltpu.VMEM((_G, D), jnp.float32),
            pltpu.VMEM((3, 16), jnp.float32),
            pltpu.SemaphoreType.DMA,
        ],
        compiler_params=pltpu.CompilerParams(needs_layout_passes=False),
    )
    eve = run(h, src, dst, w)
    return eve.reshape(_NPAD, D)[:N]


def _layer(x, src, dst, Wpool, bpool, dww, dwb, Weve, Wself, bias, relu):
    h = _pool_matmul(x, Wpool, bpool)
    eve = _sc_eve(h, src, dst, dww, dwb)
    return _out_matmul(x, Wself, eve, Weve, bias, relu)


def kernel(x, edge_index, c1_Wpool, c1_bpool, c1_dww, c1_dwb, c1_Weve, c1_Wself, c1_bias, c2_Wpool, c2_bpool, c2_dww, c2_dwb, c2_Weve, c2_Wself, c2_bias):
    src = edge_index[0]
    dst = edge_index[1]
    h = _layer(x, src, dst, c1_Wpool, c1_bpool, c1_dww, c1_dwb, c1_Weve,
               c1_Wself, c1_bias, relu=True)
    return _layer(h, src, dst, c2_Wpool, c2_bpool, c2_dww, c2_dwb, c2_Weve,
                  c2_Wself, c2_bias, relu=False)
